# baseline (speedup 1.0000x reference)
import jax
import jax.numpy as jnp
from jax.experimental import pallas as pl
from jax.experimental.pallas import tpu as pltpu

P = 512
EMB = 32
H = 64
B = 8
R = B * P


def _lstm_block(aux_ref, ht_ref, ct_ref,
                wemb_ref, bemb_ref, wih_ref, whh_ref, bias_ref,
                hout_ref, cout_ref):
    ht = ht_ref[...].reshape(R, H)
    ct = ct_ref[...].reshape(R, H)
    aux = aux_ref[...].reshape(R, 4)
    cx = aux[:, 0:1]
    cy = aux[:, 1:2]
    m = aux[:, 2:3]

    emb = jnp.maximum(
        cx * wemb_ref[0:1, :] + cy * wemb_ref[1:2, :] + bemb_ref[...], 0.0)

    gates = (jnp.dot(emb, wih_ref[...], preferred_element_type=jnp.float32)
             + jnp.dot(ht, whh_ref[...], preferred_element_type=jnp.float32)
             + bias_ref[...])

    i_g = jax.nn.sigmoid(gates[:, 0 * H:1 * H])
    f_g = jax.nn.sigmoid(gates[:, 1 * H:2 * H])
    g_g = jnp.tanh(gates[:, 2 * H:3 * H])
    o_g = jax.nn.sigmoid(gates[:, 3 * H:4 * H])

    c_new = f_g * ct + i_g * g_g
    h_new = o_g * jnp.tanh(c_new)

    hout_ref[...] = (ht + m * (h_new - ht)).reshape(B, P, H)
    cout_ref[...] = (ct + m * (c_new - ct)).reshape(B, P, H)


def kernel(corr_index, rela_ht, rela_ct, nei_index, W_emb, b_emb, W_ih, b_ih, W_hh, b_hh):
    maskf = (nei_index > 0).astype(jnp.float32)
    aux = jnp.concatenate(
        [corr_index, maskf[:, :, None], jnp.zeros((P, P, 1), jnp.float32)],
        axis=2)

    wemb = W_emb.T
    bemb = b_emb.reshape(1, EMB)
    wih = W_ih.T
    whh = W_hh.T
    bias = (b_ih + b_hh).reshape(1, 4 * H)

    grid = (P // B,)
    spec_aux = pl.BlockSpec((B, P, 4), lambda i: (i, 0, 0))
    spec3 = pl.BlockSpec((B, P, H), lambda i: (i, 0, 0))
    full = lambda a: pl.BlockSpec(a.shape, lambda i: (0, 0))

    hout, cout = pl.pallas_call(
        _lstm_block,
        grid=grid,
        in_specs=[
            spec_aux, spec3, spec3,
            full(wemb), full(bemb), full(wih), full(whh), full(bias),
        ],
        out_specs=[spec3, spec3],
        out_shape=[
            jax.ShapeDtypeStruct((P, P, H), jnp.float32),
            jax.ShapeDtypeStruct((P, P, H), jnp.float32),
        ],
        compiler_params=pltpu.CompilerParams(
            dimension_semantics=("arbitrary",),
        ),
    )(aux, rela_ht, rela_ct, wemb, bemb, wih, whh, bias)

    return (hout, cout)


# speedup vs baseline: 6.7256x; 6.7256x over previous
import jax
import jax.numpy as jnp
from jax.experimental import pallas as pl
from jax.experimental.pallas import tpu as pltpu

P = 512
EMB = 32
H = 64
BP = 8


def _lstm_block(corrT_ref, htT_ref, ctT_ref, nei_ref,
                wemb_ref, bembT_ref, wih_ref, whh_ref, biasT_ref,
                houtT_ref, coutT_ref):
    wx = wemb_ref[:, 0:1]
    wy = wemb_ref[:, 1:2]
    bembT = bembT_ref[...]
    wih = wih_ref[...]
    whh = whh_ref[...]
    biasT = biasT_ref[...]

    for j in range(BP):
        htj = htT_ref[j]
        ctj = ctT_ref[j]
        cx = corrT_ref[j, 0:1, :]
        cy = corrT_ref[j, 1:2, :]
        m = (nei_ref[j:j + 1, :] > 0).astype(jnp.float32)

        embT = jnp.maximum(wx * cx + wy * cy + bembT, 0.0)
        gates = (jnp.dot(wih, embT, preferred_element_type=jnp.float32)
                 + jnp.dot(whh, htj, preferred_element_type=jnp.float32)
                 + biasT)

        i_g = jax.nn.sigmoid(gates[0 * H:1 * H])
        f_g = jax.nn.sigmoid(gates[1 * H:2 * H])
        g_g = jnp.tanh(gates[2 * H:3 * H])
        o_g = jax.nn.sigmoid(gates[3 * H:4 * H])

        c_new = f_g * ctj + i_g * g_g
        h_new = o_g * jnp.tanh(c_new)

        houtT_ref[j] = htj + m * (h_new - htj)
        coutT_ref[j] = ctj + m * (c_new - ctj)


def kernel(corr_index, rela_ht, rela_ct, nei_index, W_emb, b_emb, W_ih, b_ih, W_hh, b_hh):
    corrT = corr_index.transpose(0, 2, 1)
    htT = rela_ht.transpose(0, 2, 1)
    ctT = rela_ct.transpose(0, 2, 1)

    bembT = b_emb.reshape(EMB, 1)
    biasT = (b_ih + b_hh).reshape(4 * H, 1)

    grid = (P // BP,)
    spec_corr = pl.BlockSpec((BP, 2, P), lambda i: (i, 0, 0))
    spec_state = pl.BlockSpec((BP, H, P), lambda i: (i, 0, 0))
    spec_nei = pl.BlockSpec((BP, P), lambda i: (i, 0))
    full = lambda a: pl.BlockSpec(a.shape, lambda i: (0, 0))

    houtT, coutT = pl.pallas_call(
        _lstm_block,
        grid=grid,
        in_specs=[
            spec_corr, spec_state, spec_state, spec_nei,
            full(W_emb), full(bembT), full(W_ih), full(W_hh), full(biasT),
        ],
        out_specs=[spec_state, spec_state],
        out_shape=[
            jax.ShapeDtypeStruct((P, H, P), jnp.float32),
            jax.ShapeDtypeStruct((P, H, P), jnp.float32),
        ],
        compiler_params=pltpu.CompilerParams(
            dimension_semantics=("arbitrary",),
        ),
    )(corrT, htT, ctT, nei_index, W_emb, bembT, W_ih, W_hh, biasT)

    return (houtT.transpose(0, 2, 1), coutT.transpose(0, 2, 1))


# speedup vs baseline: 6.7770x; 1.0076x over previous
import jax
import jax.numpy as jnp
from jax.experimental import pallas as pl
from jax.experimental.pallas import tpu as pltpu

P = 512
EMB = 32
H = 64
BP = 8


def _lstm_block(corrT_ref, htT_ref, ctT_ref, nei_ref,
                wemb_ref, bembT_ref, wih_ref, whh_ref, biasT_ref,
                houtT_ref, coutT_ref):
    wx = wemb_ref[:, 0:1]
    wy = wemb_ref[:, 1:2]
    bembT = bembT_ref[...]
    wih = wih_ref[...]
    whh = whh_ref[...]
    biasT = biasT_ref[...]

    for j in range(BP):
        htj = htT_ref[j]
        ctj = ctT_ref[j]
        cx = corrT_ref[j, 0:1, :]
        cy = corrT_ref[j, 1:2, :]
        m = nei_ref[j:j + 1, :] > 0

        embT = jnp.maximum(wx * cx + wy * cy + bembT, 0.0)
        gates = (jnp.dot(wih, embT, preferred_element_type=jnp.float32)
                 + jnp.dot(whh, htj, preferred_element_type=jnp.float32)
                 + biasT)

        t = jnp.tanh(0.5 * gates[0 * H:3 * H])
        i_g = 0.5 + 0.5 * t[0 * H:1 * H]
        f_g = 0.5 + 0.5 * t[1 * H:2 * H]
        o_g = 0.5 + 0.5 * t[2 * H:3 * H]
        g_g = jnp.tanh(gates[3 * H:4 * H])

        c_new = f_g * ctj + i_g * g_g
        h_new = o_g * jnp.tanh(c_new)

        houtT_ref[j] = jnp.where(m, h_new, htj)
        coutT_ref[j] = jnp.where(m, c_new, ctj)


def kernel(corr_index, rela_ht, rela_ct, nei_index, W_emb, b_emb, W_ih, b_ih, W_hh, b_hh):
    corrT = corr_index.transpose(0, 2, 1)
    htT = rela_ht.transpose(0, 2, 1)
    ctT = rela_ct.transpose(0, 2, 1)

    def _reord(w):
        return jnp.concatenate([w[0:2 * H], w[3 * H:4 * H], w[2 * H:3 * H]], axis=0)

    wih = _reord(W_ih)
    whh = _reord(W_hh)
    bembT = b_emb.reshape(EMB, 1)
    biasT = _reord((b_ih + b_hh).reshape(4 * H, 1))

    grid = (P // BP,)
    spec_corr = pl.BlockSpec((BP, 2, P), lambda i: (i, 0, 0))
    spec_state = pl.BlockSpec((BP, H, P), lambda i: (i, 0, 0))
    spec_nei = pl.BlockSpec((BP, P), lambda i: (i, 0))
    full = lambda a: pl.BlockSpec(a.shape, lambda i: (0, 0))

    houtT, coutT = pl.pallas_call(
        _lstm_block,
        grid=grid,
        in_specs=[
            spec_corr, spec_state, spec_state, spec_nei,
            full(W_emb), full(bembT), full(wih), full(whh), full(biasT),
        ],
        out_specs=[spec_state, spec_state],
        out_shape=[
            jax.ShapeDtypeStruct((P, H, P), jnp.float32),
            jax.ShapeDtypeStruct((P, H, P), jnp.float32),
        ],
        compiler_params=pltpu.CompilerParams(
            dimension_semantics=("arbitrary",),
        ),
    )(corrT, htT, ctT, nei_index, W_emb, bembT, wih, whh, biasT)

    return (houtT.transpose(0, 2, 1), coutT.transpose(0, 2, 1))


# speedup vs baseline: 7.9479x; 1.1728x over previous
import jax
import jax.numpy as jnp
from jax.experimental import pallas as pl
from jax.experimental.pallas import tpu as pltpu

P = 512
EMB = 32
H = 64
BP = 16


def _lstm_block(corrT_ref, htT_ref, ctT_ref, nei_ref,
                wemb_ref, bembT_ref, wih_ref, whh_ref, biasT_ref,
                houtT_ref, coutT_ref):
    wx = wemb_ref[:, 0:1]
    wy = wemb_ref[:, 1:2]
    bembT = bembT_ref[...]
    wih = wih_ref[...]
    whh = whh_ref[...]
    biasT = biasT_ref[...]

    for j in range(BP):
        htj = htT_ref[j]
        ctj = ctT_ref[j]
        cx = corrT_ref[j, 0:1, :]
        cy = corrT_ref[j, 1:2, :]
        m = nei_ref[j:j + 1, :] > 0

        embT = jnp.maximum(wx * cx + wy * cy + bembT, 0.0)
        gates = (jnp.dot(wih, embT, preferred_element_type=jnp.float32)
                 + jnp.dot(whh, htj, preferred_element_type=jnp.float32)
                 + biasT)

        t = jnp.tanh(0.5 * gates[0 * H:3 * H])
        i_g = 0.5 + 0.5 * t[0 * H:1 * H]
        f_g = 0.5 + 0.5 * t[1 * H:2 * H]
        o_g = 0.5 + 0.5 * t[2 * H:3 * H]
        g_g = jnp.tanh(gates[3 * H:4 * H])

        c_new = f_g * ctj + i_g * g_g
        h_new = o_g * jnp.tanh(c_new)

        houtT_ref[j] = jnp.where(m, h_new, htj)
        coutT_ref[j] = jnp.where(m, c_new, ctj)


def kernel(corr_index, rela_ht, rela_ct, nei_index, W_emb, b_emb, W_ih, b_ih, W_hh, b_hh):
    corrT = corr_index.transpose(0, 2, 1)
    htT = rela_ht.transpose(0, 2, 1)
    ctT = rela_ct.transpose(0, 2, 1)

    def _reord(w):
        return jnp.concatenate([w[0:2 * H], w[3 * H:4 * H], w[2 * H:3 * H]], axis=0)

    wih = _reord(W_ih)
    whh = _reord(W_hh)
    bembT = b_emb.reshape(EMB, 1)
    biasT = _reord((b_ih + b_hh).reshape(4 * H, 1))

    grid = (P // BP,)
    spec_corr = pl.BlockSpec((BP, 2, P), lambda i: (i, 0, 0))
    spec_state = pl.BlockSpec((BP, H, P), lambda i: (i, 0, 0))
    spec_nei = pl.BlockSpec((BP, P), lambda i: (i, 0))
    full = lambda a: pl.BlockSpec(a.shape, lambda i: (0, 0))

    houtT, coutT = pl.pallas_call(
        _lstm_block,
        grid=grid,
        in_specs=[
            spec_corr, spec_state, spec_state, spec_nei,
            full(W_emb), full(bembT), full(wih), full(whh), full(biasT),
        ],
        out_specs=[spec_state, spec_state],
        out_shape=[
            jax.ShapeDtypeStruct((P, H, P), jnp.float32),
            jax.ShapeDtypeStruct((P, H, P), jnp.float32),
        ],
        compiler_params=pltpu.CompilerParams(
            dimension_semantics=("arbitrary",),
        ),
    )(corrT, htT, ctT, nei_index, W_emb, bembT, wih, whh, biasT)

    return (houtT.transpose(0, 2, 1), coutT.transpose(0, 2, 1))


# speedup vs baseline: 8.6493x; 1.0882x over previous
import jax
import jax.numpy as jnp
from jax.experimental import pallas as pl
from jax.experimental.pallas import tpu as pltpu

P = 512
EMB = 32
H = 64
BP = 32


def _lstm_block(corrT_ref, htT_ref, ctT_ref, nei_ref,
                wemb_ref, bembT_ref, wih_ref, whh_ref, biasT_ref,
                houtT_ref, coutT_ref):
    wx = wemb_ref[:, 0:1]
    wy = wemb_ref[:, 1:2]
    bembT = bembT_ref[...]
    wih = wih_ref[...]
    whh = whh_ref[...]
    biasT = biasT_ref[...]

    for j in range(BP):
        htj = htT_ref[j]
        ctj = ctT_ref[j]
        cx = corrT_ref[j, 0:1, :]
        cy = corrT_ref[j, 1:2, :]
        m = nei_ref[j:j + 1, :] > 0

        embT = jnp.maximum(wx * cx + wy * cy + bembT, 0.0)
        gates = (jnp.dot(wih, embT, preferred_element_type=jnp.float32)
                 + jnp.dot(whh, htj, preferred_element_type=jnp.float32)
                 + biasT)

        t = jnp.tanh(0.5 * gates[0 * H:3 * H])
        i_g = 0.5 + 0.5 * t[0 * H:1 * H]
        f_g = 0.5 + 0.5 * t[1 * H:2 * H]
        o_g = 0.5 + 0.5 * t[2 * H:3 * H]
        g_g = jnp.tanh(gates[3 * H:4 * H])

        c_new = f_g * ctj + i_g * g_g
        h_new = o_g * jnp.tanh(c_new)

        houtT_ref[j] = jnp.where(m, h_new, htj)
        coutT_ref[j] = jnp.where(m, c_new, ctj)


def kernel(corr_index, rela_ht, rela_ct, nei_index, W_emb, b_emb, W_ih, b_ih, W_hh, b_hh):
    corrT = corr_index.transpose(0, 2, 1)
    htT = rela_ht.transpose(0, 2, 1)
    ctT = rela_ct.transpose(0, 2, 1)

    def _reord(w):
        return jnp.concatenate([w[0:2 * H], w[3 * H:4 * H], w[2 * H:3 * H]], axis=0)

    wih = _reord(W_ih)
    whh = _reord(W_hh)
    bembT = b_emb.reshape(EMB, 1)
    biasT = _reord((b_ih + b_hh).reshape(4 * H, 1))

    grid = (P // BP,)
    spec_corr = pl.BlockSpec((BP, 2, P), lambda i: (i, 0, 0))
    spec_state = pl.BlockSpec((BP, H, P), lambda i: (i, 0, 0))
    spec_nei = pl.BlockSpec((BP, P), lambda i: (i, 0))
    full = lambda a: pl.BlockSpec(a.shape, lambda i: (0, 0))

    houtT, coutT = pl.pallas_call(
        _lstm_block,
        grid=grid,
        in_specs=[
            spec_corr, spec_state, spec_state, spec_nei,
            full(W_emb), full(bembT), full(wih), full(whh), full(biasT),
        ],
        out_specs=[spec_state, spec_state],
        out_shape=[
            jax.ShapeDtypeStruct((P, H, P), jnp.float32),
            jax.ShapeDtypeStruct((P, H, P), jnp.float32),
        ],
        compiler_params=pltpu.CompilerParams(
            dimension_semantics=("arbitrary",),
        ),
    )(corrT, htT, ctT, nei_index, W_emb, bembT, wih, whh, biasT)

    return (houtT.transpose(0, 2, 1), coutT.transpose(0, 2, 1))


# speedup vs baseline: 9.0776x; 1.0495x over previous
import jax
import jax.numpy as jnp
from jax.experimental import pallas as pl
from jax.experimental.pallas import tpu as pltpu

P = 512
EMB = 32
H = 64
BP = 32

_DN0 = (((0,), (0,)), ((), ()))


def _lstm_block(corrT_ref, htT_ref, ctT_ref, nei_ref,
                wembT_ref, bembT_ref, wihT_ref, whhT_ref, biasT_ref,
                houtT_ref, coutT_ref):
    wemb = wembT_ref[...].T
    wx = wemb[:, 0:1]
    wy = wemb[:, 1:2]
    bembT = bembT_ref[...]
    wihT = wihT_ref[...]
    whhT = whhT_ref[...]
    biasT = biasT_ref[...]

    for j in range(BP):
        htj = htT_ref[j]
        ctj = ctT_ref[j]
        cx = corrT_ref[j, 0:1, :]
        cy = corrT_ref[j, 1:2, :]
        m = nei_ref[j:j + 1, :] > 0

        embT = jnp.maximum(wx * cx + wy * cy + bembT, 0.0)
        gates = (jax.lax.dot_general(wihT, embT, _DN0,
                                     preferred_element_type=jnp.float32)
                 + jax.lax.dot_general(whhT, htj, _DN0,
                                       preferred_element_type=jnp.float32)
                 + biasT)

        t_if = jnp.tanh(0.5 * gates[0 * H:2 * H])
        i_g = 0.5 + 0.5 * t_if[0 * H:1 * H]
        f_g = 0.5 + 0.5 * t_if[1 * H:2 * H]
        g_g = jnp.tanh(gates[2 * H:3 * H])
        o_g = 0.5 + 0.5 * jnp.tanh(0.5 * gates[3 * H:4 * H])

        c_new = f_g * ctj + i_g * g_g
        h_new = o_g * jnp.tanh(c_new)

        houtT_ref[j] = jnp.where(m, h_new, htj)
        coutT_ref[j] = jnp.where(m, c_new, ctj)


def kernel(corr_index, rela_ht, rela_ct, nei_index, W_emb, b_emb, W_ih, b_ih, W_hh, b_hh):
    corrT = corr_index.transpose(0, 2, 1)
    htT = rela_ht.transpose(0, 2, 1)
    ctT = rela_ct.transpose(0, 2, 1)

    wembT = W_emb.T
    wihT = W_ih.T
    whhT = W_hh.T
    bembT = b_emb.reshape(EMB, 1)
    biasT = (b_ih + b_hh).reshape(4 * H, 1)

    grid = (P // BP,)
    spec_corr = pl.BlockSpec((BP, 2, P), lambda i: (i, 0, 0))
    spec_state = pl.BlockSpec((BP, H, P), lambda i: (i, 0, 0))
    spec_nei = pl.BlockSpec((BP, P), lambda i: (i, 0))
    full = lambda a: pl.BlockSpec(a.shape, lambda i: (0, 0))

    houtT, coutT = pl.pallas_call(
        _lstm_block,
        grid=grid,
        in_specs=[
            spec_corr, spec_state, spec_state, spec_nei,
            full(wembT), full(bembT), full(wihT), full(whhT), full(biasT),
        ],
        out_specs=[spec_state, spec_state],
        out_shape=[
            jax.ShapeDtypeStruct((P, H, P), jnp.float32),
            jax.ShapeDtypeStruct((P, H, P), jnp.float32),
        ],
        compiler_params=pltpu.CompilerParams(
            dimension_semantics=("arbitrary",),
        ),
    )(corrT, htT, ctT, nei_index, wembT, bembT, wihT, whhT, biasT)

    return (houtT.transpose(0, 2, 1), coutT.transpose(0, 2, 1))
